# baseline (speedup 1.0000x reference)
import functools

import jax
import jax.numpy as jnp
from jax import lax
from jax.experimental import pallas as pl
from jax.experimental.pallas import tpu as pltpu
from jax.experimental.pallas import tpu_sc as plsc

NSUB = 16
NCORE = 2
CHUNK = 2048
NBUF = 3
EALIGN = NSUB * CHUNK * NBUF


def _sc_segment_columns(npad, n_edges):
    e_main = (n_edges // EALIGN) * EALIGN
    tail = n_edges - e_main
    assert tail % CHUNK == 0 and tail // CHUNK <= NSUB
    n_tail = tail // CHUNK
    ept = e_main // NSUB
    n_chunks = ept // CHUNK
    sl = npad // NSUB
    mesh = plsc.VectorSubcoreMesh(core_axis_name="c", subcore_axis_name="s")

    buf_types = []
    for _ in range(NBUF):
        buf_types += [
            pltpu.VMEM((CHUNK,), jnp.int32),
            pltpu.VMEM((CHUNK,), jnp.float32),
            pltpu.VMEM((CHUNK,), jnp.float32),
            pltpu.VMEM((CHUNK,), jnp.int32),
        ]

    @functools.partial(
        pl.kernel,
        out_type=[jax.ShapeDtypeStruct((npad,), jnp.float32)] * NCORE,
        name="sc_gather_segsum",
        mesh=mesh,
        compiler_params=pltpu.CompilerParams(needs_layout_passes=False),
        scratch_types=[pltpu.VMEM((npad,), jnp.float32)]
        + buf_types
        + [pltpu.SemaphoreType.DMA] * (2 * NBUF)
        + [pltpu.VMEM_SHARED((npad,), jnp.float32)],
    )
    def kern(h0_hbm, h1_hbm, src_hbm, wgt_hbm, tgt_hbm,
             acc0_hbm, acc1_hbm, col_v, *rest):
        bufs = [tuple(rest[4 * i:4 * i + 4]) for i in range(NBUF)]
        sems_in = rest[4 * NBUF:5 * NBUF]
        sems_sc = rest[5 * NBUF:6 * NBUF]
        acc_sp = rest[6 * NBUF]
        cid = lax.axis_index("c")
        sid = lax.axis_index("s")

        nreal = h0_hbm.shape[0]
        col_dst = col_v.at[pl.ds(0, nreal)]

        @pl.when(cid == 0)
        def _():
            pltpu.async_copy(h0_hbm, col_dst, sems_in[0])

        @pl.when(cid == 1)
        def _():
            pltpu.async_copy(h1_hbm, col_dst, sems_in[0])

        zv = jnp.zeros((16,), jnp.float32)
        msg0 = bufs[0][2]

        def zero_body(i, _):
            msg0[pl.ds(i * 16, 16)] = zv
            return 0

        lax.fori_loop(0, CHUNK // 16, zero_body, 0)
        off = sid * sl
        for p in range(sl // CHUNK):
            pltpu.sync_copy(msg0, acc_sp.at[pl.ds(off + p * CHUNK, CHUNK)])
        rem = sl % CHUNK
        if rem:
            pltpu.sync_copy(msg0.at[pl.ds(0, rem)],
                            acc_sp.at[pl.ds(off + (sl // CHUNK) * CHUNK, rem)])
        pltpu.make_async_copy(h0_hbm, col_dst, sems_in[0]).wait()
        plsc.subcore_barrier()

        def start_inputs(c, b):
            sv, wv, mv, tv = bufs[b]
            base = pl.multiple_of(sid * ept + c * CHUNK, 128)
            pltpu.async_copy(src_hbm.at[pl.ds(base, CHUNK)], sv, sems_in[b])
            pltpu.async_copy(wgt_hbm.at[pl.ds(base, CHUNK)], wv, sems_in[b])
            pltpu.async_copy(tgt_hbm.at[pl.ds(base, CHUNK)], tv, sems_in[b])
            pltpu.async_copy(wgt_hbm.at[pl.ds(base, CHUNK)], mv, sems_in[b])

        def wait_inputs(c, b):
            sv, wv, mv, tv = bufs[b]
            base = pl.multiple_of(sid * ept + c * CHUNK, 128)
            pltpu.make_async_copy(src_hbm.at[pl.ds(base, CHUNK)], sv, sems_in[b]).wait()
            pltpu.make_async_copy(wgt_hbm.at[pl.ds(base, CHUNK)], wv, sems_in[b]).wait()
            pltpu.make_async_copy(tgt_hbm.at[pl.ds(base, CHUNK)], tv, sems_in[b]).wait()
            pltpu.make_async_copy(wgt_hbm.at[pl.ds(base, CHUNK)], mv, sems_in[b]).wait()

        def wait_scatter(b):
            _, _, mv, tv = bufs[b]
            pltpu.make_async_copy(mv, acc_sp.at[tv], sems_sc[b]).wait()

        start_inputs(0, 0)

        def body(k3, _):
            for i in range(NBUF):
                sv, wv, mv, tv = bufs[i]
                s = k3 * NBUF + i

                @pl.when(s + 1 < n_chunks)
                def _():
                    @pl.when(s >= NBUF - 1)
                    def _():
                        wait_scatter((i + 1) % NBUF)

                    start_inputs(s + 1, (i + 1) % NBUF)

                wait_inputs(s, i)

                @plsc.parallel_loop(0, CHUNK, step=16, unroll=8)
                def _(o):
                    idx = sv[pl.ds(o, 16)]
                    vals = plsc.load_gather(col_v, [idx])
                    mv[pl.ds(o, 16)] = wv[pl.ds(o, 16)] * vals

                pltpu.async_copy(mv, acc_sp.at[tv], sems_sc[i], add=True)
            return 0

        lax.fori_loop(0, n_chunks // NBUF, body, 0)
        for i in range(NBUF):
            wait_scatter(i)

        if n_tail:
            @pl.when(sid < n_tail)
            def _():
                sv, wv, mv, tv = bufs[0]
                base = pl.multiple_of(e_main + sid * CHUNK, 128)
                pltpu.sync_copy(src_hbm.at[pl.ds(base, CHUNK)], sv)
                pltpu.sync_copy(wgt_hbm.at[pl.ds(base, CHUNK)], wv)
                pltpu.sync_copy(tgt_hbm.at[pl.ds(base, CHUNK)], tv)

                @plsc.parallel_loop(0, CHUNK, step=16, unroll=8)
                def _(o):
                    idx = sv[pl.ds(o, 16)]
                    vals = plsc.load_gather(col_v, [idx])
                    mv[pl.ds(o, 16)] = wv[pl.ds(o, 16)] * vals
                pltpu.sync_copy(mv, acc_sp.at[tv], add=True)

        plsc.subcore_barrier()

        def writeout(dst_hbm):
            pieces = [(p * CHUNK, CHUNK) for p in range(sl // CHUNK)]
            if sl % CHUNK:
                pieces.append(((sl // CHUNK) * CHUNK, sl % CHUNK))
            for (po, ln) in pieces:
                pltpu.sync_copy(acc_sp.at[pl.ds(off + po, ln)], msg0.at[pl.ds(0, ln)])
                pltpu.sync_copy(msg0.at[pl.ds(0, ln)], dst_hbm.at[pl.ds(off + po, ln)])

        @pl.when(cid == 0)
        def _():
            writeout(acc0_hbm)

        @pl.when(cid == 1)
        def _():
            writeout(acc1_hbm)

    return kern


def _tc_combine(w_ref, h0, h1, a0, a1, o0, o1):
    w00 = w_ref[0]
    w01 = w_ref[1]
    w10 = w_ref[2]
    w11 = w_ref[3]
    o0[...] = h0[...] + w00 * a0[...] + w10 * a1[...]
    o1[...] = h1[...] + w01 * a0[...] + w11 * a1[...]


def kernel(hidden_state, edge_weight, source, target, W):
    n, _ = hidden_state.shape
    e = source.shape[0]
    npad = ((n + 127) // 128) * 128

    h0 = jnp.pad(hidden_state[:, 0], (0, npad - n))
    h1 = jnp.pad(hidden_state[:, 1], (0, npad - n))
    src = source.astype(jnp.int32)
    tgt = target.astype(jnp.int32)
    wgt = edge_weight.reshape(e).astype(jnp.float32)

    acc0, acc1 = _sc_segment_columns(npad, e)(h0, h1, src, wgt, tgt)

    m = npad // 128
    o0, o1 = pl.pallas_call(
        _tc_combine,
        out_shape=[jax.ShapeDtypeStruct((m, 128), jnp.float32)] * 2,
        in_specs=[
            pl.BlockSpec(memory_space=pltpu.SMEM),
            pl.BlockSpec(memory_space=pltpu.VMEM),
            pl.BlockSpec(memory_space=pltpu.VMEM),
            pl.BlockSpec(memory_space=pltpu.VMEM),
            pl.BlockSpec(memory_space=pltpu.VMEM),
        ],
        out_specs=[pl.BlockSpec(memory_space=pltpu.VMEM)] * 2,
    )(
        W.reshape(4),
        h0.reshape(m, 128),
        h1.reshape(m, 128),
        acc0.reshape(m, 128),
        acc1.reshape(m, 128),
    )
    return jnp.stack([o0.reshape(npad)[:n], o1.reshape(npad)[:n]], axis=1)


# speedup vs baseline: 1.1251x; 1.1251x over previous
import functools

import jax
import jax.numpy as jnp
from jax import lax
from jax.experimental import pallas as pl
from jax.experimental.pallas import tpu as pltpu
from jax.experimental.pallas import tpu_sc as plsc

NSUB = 16
NCORE = 2
CHUNK = 2048
NBUF = 4
EALIGN = NSUB * CHUNK


def _sc_segment_columns(npad, n_edges):
    e_main = (n_edges // EALIGN) * EALIGN
    tail = n_edges - e_main
    assert tail % CHUNK == 0 and tail // CHUNK <= NSUB
    n_tail = tail // CHUNK
    ept = e_main // NSUB
    n_chunks = ept // CHUNK
    sl = npad // NSUB
    mesh = plsc.VectorSubcoreMesh(core_axis_name="c", subcore_axis_name="s")

    buf_types = []
    for _ in range(NBUF):
        buf_types += [
            pltpu.VMEM((CHUNK,), jnp.int32),
            pltpu.VMEM((CHUNK,), jnp.float32),
            pltpu.VMEM((CHUNK,), jnp.int32),
        ]

    @functools.partial(
        pl.kernel,
        out_type=[jax.ShapeDtypeStruct((npad,), jnp.float32)] * NCORE,
        name="sc_gather_segsum",
        mesh=mesh,
        compiler_params=pltpu.CompilerParams(needs_layout_passes=False),
        scratch_types=[pltpu.VMEM((npad,), jnp.float32)]
        + buf_types
        + [pltpu.SemaphoreType.DMA] * (2 * NBUF)
        + [pltpu.VMEM_SHARED((npad,), jnp.float32)],
    )
    def kern(h0_hbm, h1_hbm, src_hbm, wgt_hbm, tgt_hbm,
             acc0_hbm, acc1_hbm, col_v, *rest):
        bufs = [tuple(rest[3 * i:3 * i + 3]) for i in range(NBUF)]
        sems_in = rest[3 * NBUF:4 * NBUF]
        sems_sc = rest[4 * NBUF:5 * NBUF]
        acc_sp = rest[5 * NBUF]
        cid = lax.axis_index("c")
        sid = lax.axis_index("s")

        nreal = h0_hbm.shape[0]
        col_dst = col_v.at[pl.ds(0, nreal)]

        @pl.when(cid == 0)
        def _():
            pltpu.async_copy(h0_hbm, col_dst, sems_in[0])

        @pl.when(cid == 1)
        def _():
            pltpu.async_copy(h1_hbm, col_dst, sems_in[0])

        zv = jnp.zeros((16,), jnp.float32)
        msg0 = bufs[0][1]

        def zero_body(i, _):
            msg0[pl.ds(i * 16, 16)] = zv
            return 0

        lax.fori_loop(0, CHUNK // 16, zero_body, 0)
        off = sid * sl
        for p in range(sl // CHUNK):
            pltpu.sync_copy(msg0, acc_sp.at[pl.ds(off + p * CHUNK, CHUNK)])
        rem = sl % CHUNK
        if rem:
            pltpu.sync_copy(msg0.at[pl.ds(0, rem)],
                            acc_sp.at[pl.ds(off + (sl // CHUNK) * CHUNK, rem)])
        pltpu.make_async_copy(h0_hbm, col_dst, sems_in[0]).wait()
        plsc.subcore_barrier()

        def start_inputs(c, b):
            sv, wv, tv = bufs[b]
            base = pl.multiple_of(sid * ept + c * CHUNK, 128)
            pltpu.async_copy(src_hbm.at[pl.ds(base, CHUNK)], sv, sems_in[b])
            pltpu.async_copy(wgt_hbm.at[pl.ds(base, CHUNK)], wv, sems_in[b])
            pltpu.async_copy(tgt_hbm.at[pl.ds(base, CHUNK)], tv, sems_in[b])

        def wait_inputs(c, b):
            sv, wv, tv = bufs[b]
            base = pl.multiple_of(sid * ept + c * CHUNK, 128)
            pltpu.make_async_copy(src_hbm.at[pl.ds(base, CHUNK)], sv, sems_in[b]).wait()
            pltpu.make_async_copy(wgt_hbm.at[pl.ds(base, CHUNK)], wv, sems_in[b]).wait()
            pltpu.make_async_copy(tgt_hbm.at[pl.ds(base, CHUNK)], tv, sems_in[b]).wait()

        def wait_scatter(b):
            _, wv, tv = bufs[b]
            pltpu.make_async_copy(wv, acc_sp.at[tv], sems_sc[b]).wait()

        def gather_multiply(sv, wv):
            @plsc.parallel_loop(0, CHUNK, step=16, unroll=8)
            def _(o):
                idx = sv[pl.ds(o, 16)]
                vals = plsc.load_gather(col_v, [idx])
                wv[pl.ds(o, 16)] = wv[pl.ds(o, 16)] * vals

        start_inputs(0, 0)
        start_inputs(1, 1)

        def body(k4, _):
            for i in range(NBUF):
                sv, wv, tv = bufs[i]
                s = k4 * NBUF + i

                @pl.when(s + 2 < n_chunks)
                def _():
                    @pl.when(s >= 2)
                    def _():
                        wait_scatter((i + 2) % NBUF)

                    start_inputs(s + 2, (i + 2) % NBUF)

                @pl.when(s < n_chunks)
                def _():
                    wait_inputs(s, i)
                    gather_multiply(sv, wv)
                    pltpu.async_copy(wv, acc_sp.at[tv], sems_sc[i], add=True)
            return 0

        lax.fori_loop(0, (n_chunks + NBUF - 1) // NBUF, body, 0)
        for i in range(NBUF):
            wait_scatter(i)

        if n_tail:
            @pl.when(sid < n_tail)
            def _():
                sv, wv, tv = bufs[0]
                base = pl.multiple_of(e_main + sid * CHUNK, 128)
                pltpu.sync_copy(src_hbm.at[pl.ds(base, CHUNK)], sv)
                pltpu.sync_copy(wgt_hbm.at[pl.ds(base, CHUNK)], wv)
                pltpu.sync_copy(tgt_hbm.at[pl.ds(base, CHUNK)], tv)
                gather_multiply(sv, wv)
                pltpu.sync_copy(wv, acc_sp.at[tv], add=True)

        plsc.subcore_barrier()

        def writeout(dst_hbm):
            pieces = [(p * CHUNK, CHUNK) for p in range(sl // CHUNK)]
            if sl % CHUNK:
                pieces.append(((sl // CHUNK) * CHUNK, sl % CHUNK))
            for (po, ln) in pieces:
                pltpu.sync_copy(acc_sp.at[pl.ds(off + po, ln)], msg0.at[pl.ds(0, ln)])
                pltpu.sync_copy(msg0.at[pl.ds(0, ln)], dst_hbm.at[pl.ds(off + po, ln)])

        @pl.when(cid == 0)
        def _():
            writeout(acc0_hbm)

        @pl.when(cid == 1)
        def _():
            writeout(acc1_hbm)

    return kern


def _tc_combine(w_ref, h0, h1, a0, a1, o0, o1):
    w00 = w_ref[0]
    w01 = w_ref[1]
    w10 = w_ref[2]
    w11 = w_ref[3]
    o0[...] = h0[...] + w00 * a0[...] + w10 * a1[...]
    o1[...] = h1[...] + w01 * a0[...] + w11 * a1[...]


def kernel(hidden_state, edge_weight, source, target, W):
    n, _ = hidden_state.shape
    e = source.shape[0]
    npad = ((n + 127) // 128) * 128

    h0 = jnp.pad(hidden_state[:, 0], (0, npad - n))
    h1 = jnp.pad(hidden_state[:, 1], (0, npad - n))
    src = source.astype(jnp.int32)
    tgt = target.astype(jnp.int32)
    wgt = edge_weight.reshape(e).astype(jnp.float32)

    acc0, acc1 = _sc_segment_columns(npad, e)(h0, h1, src, wgt, tgt)

    m = npad // 128
    o0, o1 = pl.pallas_call(
        _tc_combine,
        out_shape=[jax.ShapeDtypeStruct((m, 128), jnp.float32)] * 2,
        in_specs=[
            pl.BlockSpec(memory_space=pltpu.SMEM),
            pl.BlockSpec(memory_space=pltpu.VMEM),
            pl.BlockSpec(memory_space=pltpu.VMEM),
            pl.BlockSpec(memory_space=pltpu.VMEM),
            pl.BlockSpec(memory_space=pltpu.VMEM),
        ],
        out_specs=[pl.BlockSpec(memory_space=pltpu.VMEM)] * 2,
    )(
        W.reshape(4),
        h0.reshape(m, 128),
        h1.reshape(m, 128),
        acc0.reshape(m, 128),
        acc1.reshape(m, 128),
    )
    return jnp.stack([o0.reshape(npad)[:n], o1.reshape(npad)[:n]], axis=1)
